# baseline (speedup 1.0000x reference)
import jax
import jax.numpy as jnp
from jax import lax
from jax.experimental import pallas as pl
from jax.experimental.pallas import tpu as pltpu

SEG = 16
NSEG = 128
TOPK = 12
DBLK = 16


def _body(x_ref, o_ref):
    ones_r = jnp.ones((SEG, NSEG), jnp.float32)
    ones_l = jnp.ones((NSEG, SEG), jnp.float32)

    a_slices = []
    for s in range(DBLK):
        es = x_ref[0, s]
        gram = jax.lax.dot_general(
            es, es,
            dimension_numbers=(((1,), (1,)), ((), ())),
            preferred_element_type=jnp.float32,
            precision=jax.lax.Precision.DEFAULT,
        )
        sq = es * es
        m1 = jax.lax.dot_general(
            sq, ones_r, dimension_numbers=(((1,), (0,)), ((), ())),
            preferred_element_type=jnp.float32,
            precision=jax.lax.Precision.HIGHEST)
        m2 = jax.lax.dot_general(
            ones_l, sq, dimension_numbers=(((1,), (1,)), ((), ())),
            preferred_element_type=jnp.float32,
            precision=jax.lax.Precision.HIGHEST)
        a_slices.append(gram * jax.lax.rsqrt(m1 * m2))
    a = jnp.stack(a_slices, axis=0)

    ri = jax.lax.broadcasted_iota(jnp.int32, (NSEG, NSEG), 0)
    ci = jax.lax.broadcasted_iota(jnp.int32, (NSEG, NSEG), 1)
    diag = (ri == ci)[None]

    work = jnp.where(diag, -jnp.inf, a)
    for _ in range(TOPK - 2):
        m = jnp.max(work, axis=-2, keepdims=True)
        work = jnp.where(work == m, -jnp.inf, work)
    thresh = jnp.max(work, axis=-2, keepdims=True)
    thresh_col = thresh.reshape(DBLK, NSEG, 1)

    keep = jnp.logical_or(a >= thresh_col, diag)
    o_ref[:, 0] = jnp.where(keep, a, 0.0)


def kernel(x):
    batch, ts_dim, ts_len = x.shape
    xs = x.reshape(batch, ts_dim, NSEG, SEG)
    out = pl.pallas_call(
        _body,
        grid=(batch, ts_dim // DBLK),
        in_specs=[
            pl.BlockSpec((1, DBLK, NSEG, SEG), lambda b, do: (b, do, 0, 0)),
        ],
        out_specs=pl.BlockSpec(
            (DBLK, 1, NSEG, NSEG), lambda b, do: (do, b, 0, 0)),
        out_shape=jax.ShapeDtypeStruct(
            (ts_dim, batch, NSEG, NSEG), jnp.float32),
        compiler_params=pltpu.CompilerParams(
            dimension_semantics=("parallel", "parallel"),
        ),
    )(xs)
    return out


# speedup vs baseline: 1.4415x; 1.4415x over previous
import jax
import jax.numpy as jnp
from jax import lax
from jax.experimental import pallas as pl
from jax.experimental.pallas import tpu as pltpu

SEG = 16
NSEG = 128
TOPK = 12
DBLK = 16


def _body(x_ref, o_ref):
    a_slices = []
    for s in range(DBLK):
        es = x_ref[0, s]
        gram = jax.lax.dot_general(
            es, es,
            dimension_numbers=(((1,), (1,)), ((), ())),
            preferred_element_type=jnp.float32,
            precision=jax.lax.Precision.DEFAULT,
        )
        n2 = jnp.sum(es * es, axis=1)
        norms = jnp.sqrt(n2)
        a_slices.append(gram / (norms[:, None] * norms[None, :]))
    a = jnp.stack(a_slices, axis=0)

    ri = jax.lax.broadcasted_iota(jnp.int32, (NSEG, NSEG), 0)
    ci = jax.lax.broadcasted_iota(jnp.int32, (NSEG, NSEG), 1)
    diag = (ri == ci)[None]

    work = jnp.where(diag, -jnp.inf, a)
    for _ in range(TOPK - 2):
        m = jnp.max(work, axis=-2, keepdims=True)
        work = jnp.where(work == m, -jnp.inf, work)
    thresh = jnp.max(work, axis=-2, keepdims=True)
    thresh_col = thresh.reshape(DBLK, NSEG, 1)

    keep = jnp.logical_or(a >= thresh_col, diag)
    o_ref[:, 0] = jnp.where(keep, a, 0.0)


def kernel(x):
    batch, ts_dim, ts_len = x.shape
    xs = x.reshape(batch, ts_dim, NSEG, SEG)
    out = pl.pallas_call(
        _body,
        grid=(batch, ts_dim // DBLK),
        in_specs=[
            pl.BlockSpec((1, DBLK, NSEG, SEG), lambda b, do: (b, do, 0, 0)),
        ],
        out_specs=pl.BlockSpec(
            (DBLK, 1, NSEG, NSEG), lambda b, do: (do, b, 0, 0)),
        out_shape=jax.ShapeDtypeStruct(
            (ts_dim, batch, NSEG, NSEG), jnp.float32),
        compiler_params=pltpu.CompilerParams(
            dimension_semantics=("parallel", "parallel"),
        ),
    )(xs)
    return out


# speedup vs baseline: 1.4598x; 1.0127x over previous
import jax
import jax.numpy as jnp
from jax import lax
from jax.experimental import pallas as pl
from jax.experimental.pallas import tpu as pltpu

SEG = 16
NSEG = 128
TOPK = 12
DBLK = 16
XSPL = 8


def _body(x_ref, o_ref):
    xb = x_ref[0]
    e = xb.reshape(DBLK, NSEG, SEG)

    a_slices = []
    for s in range(DBLK):
        es = e[s]
        gram = jax.lax.dot_general(
            es, es,
            dimension_numbers=(((1,), (1,)), ((), ())),
            preferred_element_type=jnp.float32,
            precision=jax.lax.Precision.DEFAULT,
        )
        n2 = jnp.sum(es * es, axis=1)
        norms = jnp.sqrt(n2)
        a_slices.append(gram / (norms[:, None] * norms[None, :]))
    a = jnp.stack(a_slices, axis=0)

    ri = jax.lax.broadcasted_iota(jnp.int32, (NSEG, NSEG), 0)
    ci = jax.lax.broadcasted_iota(jnp.int32, (NSEG, NSEG), 1)
    diag = (ri == ci)[None]

    aV, aX = a[:XSPL], a[XSPL:]

    work = jnp.where(diag, -jnp.inf, aV)
    for _ in range(TOPK - 2):
        m = jnp.max(work, axis=-2, keepdims=True)
        work = jnp.where(work == m, -jnp.inf, work)
    threshV = jnp.max(work, axis=-2, keepdims=True)
    threshV_col = threshV.reshape(XSPL, NSEG, 1)

    work = jnp.where(diag, -jnp.inf, aX)
    for _ in range(TOPK - 2):
        m = jnp.max(work, axis=-1, keepdims=True)
        work = jnp.where(work == m, -jnp.inf, work)
    threshX = jnp.max(work, axis=-1, keepdims=True)

    keepV = jnp.logical_or(aV >= threshV_col, diag)
    keepX = jnp.logical_or(aX >= threshX, diag)
    o_ref[:XSPL, 0] = jnp.where(keepV, aV, 0.0)
    o_ref[XSPL:, 0] = jnp.where(keepX, aX, 0.0)


def kernel(x):
    batch, ts_dim, ts_len = x.shape
    out = pl.pallas_call(
        _body,
        grid=(batch, ts_dim // DBLK),
        in_specs=[
            pl.BlockSpec((1, DBLK, ts_len), lambda b, do: (b, do, 0)),
        ],
        out_specs=pl.BlockSpec(
            (DBLK, 1, NSEG, NSEG), lambda b, do: (do, b, 0, 0)),
        out_shape=jax.ShapeDtypeStruct(
            (ts_dim, batch, NSEG, NSEG), jnp.float32),
        compiler_params=pltpu.CompilerParams(
            dimension_semantics=("parallel", "parallel"),
        ),
    )(x)
    return out


# speedup vs baseline: 1.9363x; 1.3264x over previous
import jax
import jax.numpy as jnp
from jax import lax
from jax.experimental import pallas as pl
from jax.experimental.pallas import tpu as pltpu

SEG = 16
NSEG = 128
TOPK = 12
DBLK = 16


def _body(x_ref, o_ref):
    xb = x_ref[0]
    e = xb.reshape(DBLK, NSEG, SEG)

    a_slices = []
    for s in range(DBLK):
        es = e[s]
        gram = jax.lax.dot_general(
            es, es,
            dimension_numbers=(((1,), (1,)), ((), ())),
            preferred_element_type=jnp.float32,
            precision=jax.lax.Precision.DEFAULT,
        )
        n2 = jnp.sum(es * es, axis=1)
        inv = 1.0 / jnp.sqrt(n2)
        a_slices.append(gram * (inv[:, None] * inv[None, :]))
    a = jnp.stack(a_slices, axis=0)

    ri = jax.lax.broadcasted_iota(jnp.int32, (NSEG, NSEG), 0)
    ci = jax.lax.broadcasted_iota(jnp.int32, (NSEG, NSEG), 1)
    diag = (ri == ci)[None]

    work = jnp.where(diag, -jnp.inf, a)
    for _ in range(TOPK - 2):
        m = jnp.max(work, axis=-2, keepdims=True)
        work = jnp.where(work == m, -jnp.inf, work)
    thresh = jnp.max(work, axis=-2, keepdims=True)
    thresh_col = thresh.reshape(DBLK, NSEG, 1)

    keep = jnp.logical_or(a >= thresh_col, diag)
    o_ref[:, 0] = jnp.where(keep, a, 0.0)


def kernel(x):
    batch, ts_dim, ts_len = x.shape
    out = pl.pallas_call(
        _body,
        grid=(batch, ts_dim // DBLK),
        in_specs=[
            pl.BlockSpec((1, DBLK, ts_len), lambda b, do: (b, do, 0)),
        ],
        out_specs=pl.BlockSpec(
            (DBLK, 1, NSEG, NSEG), lambda b, do: (do, b, 0, 0)),
        out_shape=jax.ShapeDtypeStruct(
            (ts_dim, batch, NSEG, NSEG), jnp.float32),
        compiler_params=pltpu.CompilerParams(
            dimension_semantics=("parallel", "parallel"),
        ),
    )(x)
    return out
